# baseline (speedup 1.0000x reference)
import functools

import jax
import jax.numpy as jnp
from jax import lax
from jax.experimental import pallas as pl
from jax.experimental.pallas import tpu as pltpu
from jax.experimental.pallas import tpu_sc as plsc

_B = 4096
_D = 32
_DW = 128
_LLANE = 32
_NSUB = 3
_NC = 3000
_NCP = 3072
_MARGIN = 1.0
_BX = 1024
_BC = 768
_BD = 384
_NS1 = _B // _BX
_KPAD = 40
_NEG = -1e30


def _augment(rows, b):
    n = rows.shape[0]
    return jnp.concatenate(
        [rows + rows, b, jnp.ones((n, 1), jnp.float32),
         jnp.zeros((n, _KPAD - _D - 2), jnp.float32)], axis=1)


def _tc_body(x_ref, cfull_ref, crows_ref, cb_ref, pfull_ref,
             pdiag_ref, out_ref, caug_s, q_s):
    g = pl.program_id(0)
    ones_row = jnp.ones((1, _D), jnp.float32)

    @pl.when(g == 0)
    def _init():
        cfull = cfull_ref[...]
        c2_full = lax.dot_general(ones_row, cfull * cfull,
                                  (((1,), (1,)), ((), ())),
                                  preferred_element_type=jnp.float32)
        present = jnp.max(pfull_ref[...], axis=0, keepdims=True)
        colid = lax.broadcasted_iota(jnp.int32, (1, _NCP), 1)
        colcls = colid // _NSUB
        q_s[...] = jnp.where(present > 0.5, colcls, -1)
        negc2 = jnp.where(colid < _NC, -c2_full, _NEG)
        caug_s[...] = jnp.concatenate(
            [cfull, jnp.ones((_NCP, 1), jnp.float32),
             negc2.reshape(_NCP, 1),
             jnp.zeros((_NCP, _KPAD - _D - 2), jnp.float32)],
            axis=1)
        out_ref[0] = 0.0
        out_ref[1] = 0.0
        out_ref[2] = 0.0
        out_ref[3] = 0.0

    caug = caug_s[...]
    q = q_s[...]

    xb = x_ref[...]
    cbw = cb_ref[...]
    cbb = cbw[:, :_D]
    lab = cbw[:, _LLANE:_LLANE + 1].astype(jnp.int32)
    diff = xb - cbb
    intra = jnp.sum(diff * diff, axis=1, keepdims=True)
    x2 = jnp.sum(xb * xb, axis=1, keepdims=True)
    b = _MARGIN + intra - x2
    xaug = _augment(xb, b)
    t = lax.dot_general(xaug, caug, (((1,), (1,)), ((), ())),
                        preferred_element_type=jnp.float32)
    lcls = lab // _NSUB
    r = jnp.where(q == lcls, 0.0, jnp.maximum(t, 0.0))
    out_ref[0] += jnp.sum(intra)
    out_ref[1] += jnp.sum(r)

    cr_full = crows_ref[...]
    corr_sum = jnp.float32(0.0)
    b2_halves = []
    for h in range(_BC // _BD):
        cr = cr_full[h * _BD:(h + 1) * _BD]
        c2col = jnp.sum(cr * cr, axis=1, keepdims=True)
        base = g * _BC + h * _BD
        rowid = lax.broadcasted_iota(jnp.int32, (_BD, 1), 0) + base
        rowcls = rowid // _NSUB
        colid_d = lax.broadcasted_iota(jnp.int32, (1, _BD), 1) + base
        colcls_d = colid_d // _NSUB
        pd = jnp.max(pdiag_ref[:, h * _BD:(h + 1) * _BD], axis=0,
                     keepdims=True)
        crsq = cr * cr
        c2row_d = lax.dot_general(ones_row, crsq, (((1,), (1,)), ((), ())),
                                  preferred_element_type=jnp.float32)
        ccd = lax.dot_general(cr, cr, (((1,), (1,)), ((), ())),
                              preferred_element_type=jnp.float32)
        dd = c2col + c2row_d - 2.0 * ccd

        samecls_d = rowcls == colcls_d
        eye = rowid == colid_d
        pdb = jnp.broadcast_to(pd, (_BD, _BD))
        pcol = jnp.max(jnp.where(eye, pdb, 0.0), axis=1,
                       keepdims=True)
        colm = jnp.max(jnp.where(samecls_d & (pcol > 0.5), dd, _NEG),
                       axis=0, keepdims=True)
        colmb = jnp.broadcast_to(colm, (_BD, _BD))
        dmax = jnp.max(jnp.where(samecls_d & (pdb > 0.5), colmb, _NEG),
                       axis=1, keepdims=True)
        cnt = jnp.sum(jnp.where(samecls_d, pdb, 0.0), axis=1,
                      keepdims=True)
        care = (cnt > 1.5).astype(jnp.float32)
        w = care * pcol

        b2 = jnp.where(w > 0.5, _MARGIN + dmax - c2col, _NEG)
        b2_halves.append(b2)
        t2d = 2.0 * ccd + b2 - c2row_d
        qd = jnp.where(pd > 0.5, colcls_d, -1)
        corr = jnp.where(qd == rowcls, jnp.maximum(t2d, 0.0), 0.0)
        corr_sum = corr_sum + jnp.sum(corr)

    craug = _augment(cr_full, jnp.concatenate(b2_halves, axis=0))
    t2 = lax.dot_general(craug, caug, (((1,), (1,)), ((), ())),
                         preferred_element_type=jnp.float32)
    out_ref[2] += jnp.sum(jnp.maximum(t2, 0.0)) - corr_sum


def _s1_map(g):
    return (g, 0)


def _s2_map(g):
    return (g, 0)


def _pdiag_map(g):
    return (0, g)


_TC_KW = dict(
    grid=(_NS1,),
    in_specs=[
        pl.BlockSpec((_BX, _D), _s1_map),
        pl.BlockSpec((_NCP, _D), lambda g: (0, 0)),
        pl.BlockSpec((_BC, _D), _s2_map),
        pl.BlockSpec((_BX, _DW), _s1_map),
        pl.BlockSpec((32, _NCP), lambda g: (0, 0)),
        pl.BlockSpec((32, _BC), _pdiag_map),
    ],
    out_specs=pl.BlockSpec(memory_space=pltpu.SMEM),
    out_shape=jax.ShapeDtypeStruct((4,), jnp.float32),
    scratch_shapes=[
        pltpu.VMEM((_NCP, _KPAD), jnp.float32),
        pltpu.VMEM((1, _NCP), jnp.int32),
    ],
    compiler_params=pltpu.CompilerParams(
        dimension_semantics=("arbitrary",)),
)

_tc_call = pl.pallas_call(_tc_body, **_TC_KW)


_NW = 32
_BPW = _B // _NW


@functools.cache
def _sc_kernels():
    mesh = plsc.VectorSubcoreMesh(core_axis_name="c", subcore_axis_name="s")

    @functools.partial(
        pl.kernel,
        mesh=mesh,
        out_type=[
            jax.ShapeDtypeStruct((_NW, _NCP), jnp.float32),
            jax.ShapeDtypeStruct((_B, _DW), jnp.float32),
        ],
        scratch_types=[
            pltpu.VMEM((_BPW,), jnp.int32),
            pltpu.VMEM((_BPW, _DW), jnp.float32),
            pltpu.VMEM((_NCP,), jnp.float32),
            pltpu.SemaphoreType.DMA,
        ],
        compiler_params=pltpu.CompilerParams(needs_layout_passes=False),
    )
    def _sc_stage(labels_hbm, c_hbm, present_hbm, cb_hbm, idx_v, rows_v,
                  pbuf, sem):
        wid = lax.axis_index("s") * 2 + lax.axis_index("c")
        base = wid * _BPW
        pltpu.sync_copy(labels_hbm.at[pl.ds(base, _BPW)], idx_v)
        copy = pltpu.async_copy(c_hbm.at[idx_v], rows_v, sem)

        zero16 = jnp.zeros((16,), jnp.float32)
        for i in range(_NCP // 16):
            pbuf[pl.ds(i * 16, 16)] = zero16
        ones16 = jnp.ones((16,), jnp.float32)
        for j in range(_BPW // 16):
            plsc.store_scatter(pbuf, [idx_v[pl.ds(j * 16, 16)]], ones16)
        pltpu.sync_copy(pbuf, present_hbm.at[wid])

        copy.wait()
        lane = jnp.full((16,), _LLANE, jnp.int32)
        for j in range(_BPW // 16):
            ridx = lax.broadcasted_iota(jnp.int32, (16,), 0) + j * 16
            vals = idx_v[pl.ds(j * 16, 16)].astype(jnp.float32)
            plsc.store_scatter(rows_v, [ridx, lane], vals)
        pltpu.sync_copy(rows_v, cb_hbm.at[pl.ds(base, _BPW)])

    return _sc_stage


def _sc_part(labels, cwide):
    return _sc_kernels()(labels, cwide)


def kernel(x, labels, centers):
    c = centers.reshape(_NC, _D)
    cpad = jnp.pad(c, ((0, _NCP - _NC), (0, 0)))
    cwide = jnp.pad(c, ((0, _NCP - _NC), (0, _DW - _D)))
    presentp, cbw = _sc_part(labels, cwide)
    sums = _tc_call(x, cpad, cpad, cbw, presentp, presentp)
    intraclass = sums[0] / (_B * _D * 2.0)
    triplet = sums[1] / (2.0 * _NC * _B)
    interclass = sums[2] / (_NC * _B * 2.0)
    return (intraclass, interclass, triplet)


# speedup vs baseline: 1.0716x; 1.0716x over previous
import functools

import jax
import jax.numpy as jnp
from jax import lax
from jax.experimental import pallas as pl
from jax.experimental.pallas import tpu as pltpu
from jax.experimental.pallas import tpu_sc as plsc

_B = 4096
_D = 32
_DW = 128
_LLANE = 32
_NSUB = 3
_NC = 3000
_NCP = 3072
_MARGIN = 1.0
_BX = 1024
_BC = 768
_BD = 384
_NS1 = _B // _BX
_KPAD = 40
_NEG = -1e30


def _augment(rows, b):
    n = rows.shape[0]
    parts = [rows + rows, b, jnp.ones((n, 1), jnp.float32)]
    if _KPAD > _D + 2:
        parts.append(jnp.zeros((n, _KPAD - _D - 2), jnp.float32))
    return jnp.concatenate(parts, axis=1)


def _tc_body(x_ref, cfull_ref, crows_ref, cb_ref, pfull_ref,
             pdiag_ref, out_ref, caug_s, q_s):
    g = pl.program_id(0)
    ones_row = jnp.ones((1, _D), jnp.float32)

    @pl.when(g == 0)
    def _init():
        cfull = cfull_ref[...]
        c2_full = lax.dot_general(ones_row, cfull * cfull,
                                  (((1,), (1,)), ((), ())),
                                  preferred_element_type=jnp.float32)
        present = jnp.max(pfull_ref[...], axis=0, keepdims=True)
        colid = lax.broadcasted_iota(jnp.int32, (1, _NCP), 1)
        colcls = colid // _NSUB
        q_s[...] = jnp.where(present > 0.5, colcls, -1)
        negc2 = jnp.where(colid < _NC, -c2_full, _NEG)
        cparts = [cfull, jnp.ones((_NCP, 1), jnp.float32),
                  negc2.reshape(_NCP, 1)]
        if _KPAD > _D + 2:
            cparts.append(jnp.zeros((_NCP, _KPAD - _D - 2), jnp.float32))
        caug_s[...] = jnp.concatenate(cparts, axis=1)
        out_ref[0] = 0.0
        out_ref[1] = 0.0
        out_ref[2] = 0.0
        out_ref[3] = 0.0

    caug = caug_s[...]
    q = q_s[...]

    xb = x_ref[...]
    cbw = cb_ref[...]
    cbb = cbw[:, :_D]
    lab = cbw[:, _LLANE:_LLANE + 1].astype(jnp.int32)
    diff = xb - cbb
    intra = jnp.sum(diff * diff, axis=1, keepdims=True)
    x2 = jnp.sum(xb * xb, axis=1, keepdims=True)
    b = _MARGIN + intra - x2
    xaug = _augment(xb, b)
    t = lax.dot_general(xaug, caug, (((1,), (1,)), ((), ())),
                        preferred_element_type=jnp.float32)
    lcls = lab // _NSUB
    r = jnp.where(q == lcls, 0.0, jnp.maximum(t, 0.0))
    out_ref[0] += jnp.sum(intra)
    out_ref[1] += jnp.sum(r)

    cr_full = crows_ref[...]
    corr_sum = jnp.float32(0.0)
    for h in range(_BC // _BD):
        cr = cr_full[h * _BD:(h + 1) * _BD]
        c2col = jnp.sum(cr * cr, axis=1, keepdims=True)
        base = g * _BC + h * _BD
        rowid = lax.broadcasted_iota(jnp.int32, (_BD, 1), 0) + base
        rowcls = rowid // _NSUB
        colid_d = lax.broadcasted_iota(jnp.int32, (1, _BD), 1) + base
        colcls_d = colid_d // _NSUB
        pd = jnp.max(pdiag_ref[:, h * _BD:(h + 1) * _BD], axis=0,
                     keepdims=True)
        crsq = cr * cr
        c2row_d = lax.dot_general(ones_row, crsq, (((1,), (1,)), ((), ())),
                                  preferred_element_type=jnp.float32)
        ccd = lax.dot_general(cr, cr, (((1,), (1,)), ((), ())),
                              preferred_element_type=jnp.float32)
        dd = c2col + c2row_d - 2.0 * ccd

        samecls_d = rowcls == colcls_d
        eye = rowid == colid_d
        pdb = jnp.broadcast_to(pd, (_BD, _BD))
        pcol = jnp.max(jnp.where(eye, pdb, 0.0), axis=1,
                       keepdims=True)
        colm = jnp.max(jnp.where(samecls_d & (pcol > 0.5), dd, _NEG),
                       axis=0, keepdims=True)
        colmb = jnp.broadcast_to(colm, (_BD, _BD))
        dmax = jnp.max(jnp.where(samecls_d & (pdb > 0.5), colmb, _NEG),
                       axis=1, keepdims=True)
        cnt = jnp.sum(jnp.where(samecls_d, pdb, 0.0), axis=1,
                      keepdims=True)
        care = (cnt > 1.5).astype(jnp.float32)
        w = care * pcol

        b2 = jnp.where(w > 0.5, _MARGIN + dmax - c2col, _NEG)
        craug = _augment(cr, b2)
        t2 = lax.dot_general(craug, caug, (((1,), (1,)), ((), ())),
                             preferred_element_type=jnp.float32)
        t2d = 2.0 * ccd + b2 - c2row_d
        qd = jnp.where(pd > 0.5, colcls_d, -1)
        corr = jnp.where(qd == rowcls, jnp.maximum(t2d, 0.0), 0.0)
        corr_sum = corr_sum + (jnp.sum(jnp.maximum(t2, 0.0)) - jnp.sum(corr))

    out_ref[2] += corr_sum


def _s1_map(g):
    return (g, 0)


def _s2_map(g):
    return (g, 0)


def _pdiag_map(g):
    return (0, g)


_TC_KW = dict(
    grid=(_NS1,),
    in_specs=[
        pl.BlockSpec((_BX, _D), _s1_map),
        pl.BlockSpec((_NCP, _D), lambda g: (0, 0)),
        pl.BlockSpec((_BC, _D), _s2_map),
        pl.BlockSpec((_BX, _DW), _s1_map),
        pl.BlockSpec((32, _NCP), lambda g: (0, 0)),
        pl.BlockSpec((32, _BC), _pdiag_map),
    ],
    out_specs=pl.BlockSpec(memory_space=pltpu.SMEM),
    out_shape=jax.ShapeDtypeStruct((4,), jnp.float32),
    scratch_shapes=[
        pltpu.VMEM((_NCP, _KPAD), jnp.float32),
        pltpu.VMEM((1, _NCP), jnp.int32),
    ],
    compiler_params=pltpu.CompilerParams(
        dimension_semantics=("arbitrary",)),
)

_tc_call = pl.pallas_call(_tc_body, **_TC_KW)


_NW = 32
_BPW = _B // _NW


@functools.cache
def _sc_kernels():
    mesh = plsc.VectorSubcoreMesh(core_axis_name="c", subcore_axis_name="s")

    @functools.partial(
        pl.kernel,
        mesh=mesh,
        out_type=[
            jax.ShapeDtypeStruct((_NW, _NCP), jnp.float32),
            jax.ShapeDtypeStruct((_B, _DW), jnp.float32),
        ],
        scratch_types=[
            pltpu.VMEM((_BPW,), jnp.int32),
            pltpu.VMEM((_BPW, _DW), jnp.float32),
            pltpu.VMEM((_NCP,), jnp.float32),
            pltpu.SemaphoreType.DMA,
        ],
        compiler_params=pltpu.CompilerParams(needs_layout_passes=False),
    )
    def _sc_stage(labels_hbm, c_hbm, present_hbm, cb_hbm, idx_v, rows_v,
                  pbuf, sem):
        wid = lax.axis_index("s") * 2 + lax.axis_index("c")
        base = wid * _BPW
        pltpu.sync_copy(labels_hbm.at[pl.ds(base, _BPW)], idx_v)
        copy = pltpu.async_copy(c_hbm.at[idx_v], rows_v, sem)

        zero16 = jnp.zeros((16,), jnp.float32)
        for i in range(_NCP // 16):
            pbuf[pl.ds(i * 16, 16)] = zero16
        ones16 = jnp.ones((16,), jnp.float32)
        for j in range(_BPW // 16):
            plsc.store_scatter(pbuf, [idx_v[pl.ds(j * 16, 16)]], ones16)
        pltpu.sync_copy(pbuf, present_hbm.at[wid])

        copy.wait()
        lane = jnp.full((16,), _LLANE, jnp.int32)
        for j in range(_BPW // 16):
            ridx = lax.broadcasted_iota(jnp.int32, (16,), 0) + j * 16
            vals = idx_v[pl.ds(j * 16, 16)].astype(jnp.float32)
            plsc.store_scatter(rows_v, [ridx, lane], vals)
        pltpu.sync_copy(rows_v, cb_hbm.at[pl.ds(base, _BPW)])

    return _sc_stage


def _sc_part(labels, cwide):
    return _sc_kernels()(labels, cwide)


def kernel(x, labels, centers):
    c = centers.reshape(_NC, _D)
    cpad = jnp.pad(c, ((0, _NCP - _NC), (0, 0)))
    cwide = jnp.pad(c, ((0, _NCP - _NC), (0, _DW - _D)))
    presentp, cbw = _sc_part(labels, cwide)
    sums = _tc_call(x, cpad, cpad, cbw, presentp, presentp)
    intraclass = sums[0] / (_B * _D * 2.0)
    triplet = sums[1] / (2.0 * _NC * _B)
    interclass = sums[2] / (_NC * _B * 2.0)
    return (intraclass, interclass, triplet)


# speedup vs baseline: 1.2251x; 1.1433x over previous
import functools

import jax
import jax.numpy as jnp
from jax import lax
from jax.experimental import pallas as pl
from jax.experimental.pallas import tpu as pltpu
from jax.experimental.pallas import tpu_sc as plsc

_B = 4096
_D = 32
_DW = 128
_LLANE = 32
_NSUB = 3
_NC = 3000
_NCP = 3072
_MARGIN = 1.0
_BX = 1024
_BC = 768
_BD = 384
_NS1 = _B // _BX
_KPAD = 40
_NEG = -1e30


def _augment(rows, b):
    n = rows.shape[0]
    parts = [rows + rows, b, jnp.ones((n, 1), jnp.float32)]
    if _KPAD > _D + 2:
        parts.append(jnp.zeros((n, _KPAD - _D - 2), jnp.float32))
    return jnp.concatenate(parts, axis=1)


def _tc_body(x_ref, cfull_ref, crows_ref, cb_ref, pfull_ref,
             pdiag_ref, out_ref, caug_s, q_s, acc_s):
    g = pl.program_id(0)
    ones_row = jnp.ones((1, _D), jnp.float32)

    @pl.when(g == 0)
    def _init():
        cfull = cfull_ref[...]
        c2_full = lax.dot_general(ones_row, cfull * cfull,
                                  (((1,), (1,)), ((), ())),
                                  preferred_element_type=jnp.float32)
        present = jnp.max(pfull_ref[...], axis=0, keepdims=True)
        colid = lax.broadcasted_iota(jnp.int32, (1, _NCP), 1)
        colcls = colid // _NSUB
        q_s[...] = jnp.where(present > 0.5, colcls, -1)
        negc2 = jnp.where(colid < _NC, -c2_full, _NEG)
        cparts = [cfull, jnp.ones((_NCP, 1), jnp.float32),
                  negc2.reshape(_NCP, 1)]
        if _KPAD > _D + 2:
            cparts.append(jnp.zeros((_NCP, _KPAD - _D - 2), jnp.float32))
        caug_s[...] = jnp.concatenate(cparts, axis=1)
        out_ref[0] = 0.0
        out_ref[1] = 0.0
        out_ref[2] = 0.0
        out_ref[3] = 0.0

    caug = caug_s[...]
    q = q_s[...]

    xb = x_ref[...]
    cbw = cb_ref[...]
    cbb = cbw[:, :_D]
    lab = cbw[:, _LLANE:_LLANE + 1].astype(jnp.int32)
    diff = xb - cbb
    intra = jnp.sum(diff * diff, axis=1, keepdims=True)
    x2 = jnp.sum(xb * xb, axis=1, keepdims=True)
    b = _MARGIN + intra - x2
    xaug = _augment(xb, b)
    t = lax.dot_general(xaug, caug, (((1,), (1,)), ((), ())),
                        preferred_element_type=jnp.float32)
    lcls = lab // _NSUB
    r = jnp.where(q == lcls, 0.0, jnp.maximum(t, 0.0))
    out_ref[0] += jnp.sum(intra)
    rpart = jnp.sum(r, axis=0, keepdims=True)
    @pl.when(g == 0)
    def _acc_init1():
        acc_s[0:1, :] = rpart
    @pl.when(g > 0)
    def _acc_add1():
        acc_s[0:1, :] += rpart

    cr_full = crows_ref[...]
    corr_sum = jnp.float32(0.0)
    for h in range(_BC // _BD):
        cr = cr_full[h * _BD:(h + 1) * _BD]
        c2col = jnp.sum(cr * cr, axis=1, keepdims=True)
        base = g * _BC + h * _BD
        rowid = lax.broadcasted_iota(jnp.int32, (_BD, 1), 0) + base
        rowcls = rowid // _NSUB
        colid_d = lax.broadcasted_iota(jnp.int32, (1, _BD), 1) + base
        colcls_d = colid_d // _NSUB
        pd = jnp.max(pdiag_ref[:, h * _BD:(h + 1) * _BD], axis=0,
                     keepdims=True)
        crsq = cr * cr
        c2row_d = lax.dot_general(ones_row, crsq, (((1,), (1,)), ((), ())),
                                  preferred_element_type=jnp.float32)
        ccd = lax.dot_general(cr, cr, (((1,), (1,)), ((), ())),
                              preferred_element_type=jnp.float32)
        dd = c2col + c2row_d - 2.0 * ccd

        samecls_d = rowcls == colcls_d
        eye = rowid == colid_d
        pdb = jnp.broadcast_to(pd, (_BD, _BD))
        pcol = jnp.max(jnp.where(eye, pdb, 0.0), axis=1,
                       keepdims=True)
        colm = jnp.max(jnp.where(samecls_d & (pcol > 0.5), dd, _NEG),
                       axis=0, keepdims=True)
        colmb = jnp.broadcast_to(colm, (_BD, _BD))
        dmax = jnp.max(jnp.where(samecls_d & (pdb > 0.5), colmb, _NEG),
                       axis=1, keepdims=True)
        cnt = jnp.sum(jnp.where(samecls_d, pdb, 0.0), axis=1,
                      keepdims=True)
        care = (cnt > 1.5).astype(jnp.float32)
        w = care * pcol

        b2 = jnp.where(w > 0.5, _MARGIN + dmax - c2col, _NEG)
        craug = _augment(cr, b2)
        t2 = lax.dot_general(craug, caug, (((1,), (1,)), ((), ())),
                             preferred_element_type=jnp.float32)
        t2d = 2.0 * ccd + b2 - c2row_d
        qd = jnp.where(pd > 0.5, colcls_d, -1)
        corr = jnp.where(qd == rowcls, jnp.maximum(t2d, 0.0), 0.0)
        corr_sum = corr_sum - jnp.sum(corr)
        r2part = jnp.sum(jnp.maximum(t2, 0.0), axis=0, keepdims=True)
        if h == 0:
            r2acc = r2part
        else:
            r2acc = r2acc + r2part

    out_ref[2] += corr_sum
    @pl.when(g == 0)
    def _acc_init2():
        acc_s[1:2, :] = r2acc
    @pl.when(g > 0)
    def _acc_add2():
        acc_s[1:2, :] += r2acc

    @pl.when(g == _NS1 - 1)
    def _final():
        out_ref[1] += jnp.sum(acc_s[0:1, :])
        out_ref[2] += jnp.sum(acc_s[1:2, :])


def _s1_map(g):
    return (g, 0)


def _s2_map(g):
    return (g, 0)


def _pdiag_map(g):
    return (0, g)


_TC_KW = dict(
    grid=(_NS1,),
    in_specs=[
        pl.BlockSpec((_BX, _D), _s1_map),
        pl.BlockSpec((_NCP, _D), lambda g: (0, 0)),
        pl.BlockSpec((_BC, _D), _s2_map),
        pl.BlockSpec((_BX, _DW), _s1_map),
        pl.BlockSpec((32, _NCP), lambda g: (0, 0)),
        pl.BlockSpec((32, _BC), _pdiag_map),
    ],
    out_specs=pl.BlockSpec(memory_space=pltpu.SMEM),
    out_shape=jax.ShapeDtypeStruct((4,), jnp.float32),
    scratch_shapes=[
        pltpu.VMEM((_NCP, _KPAD), jnp.float32),
        pltpu.VMEM((1, _NCP), jnp.int32),
        pltpu.VMEM((2, _NCP), jnp.float32),
    ],
    compiler_params=pltpu.CompilerParams(
        dimension_semantics=("arbitrary",)),
)

_tc_call = pl.pallas_call(_tc_body, **_TC_KW)


_NW = 32
_BPW = _B // _NW


@functools.cache
def _sc_kernels():
    mesh = plsc.VectorSubcoreMesh(core_axis_name="c", subcore_axis_name="s")

    @functools.partial(
        pl.kernel,
        mesh=mesh,
        out_type=[
            jax.ShapeDtypeStruct((_NW, _NCP), jnp.float32),
            jax.ShapeDtypeStruct((_B, _DW), jnp.float32),
        ],
        scratch_types=[
            pltpu.VMEM((_BPW,), jnp.int32),
            pltpu.VMEM((_BPW, _DW), jnp.float32),
            pltpu.VMEM((_NCP,), jnp.float32),
            pltpu.SemaphoreType.DMA,
        ],
        compiler_params=pltpu.CompilerParams(needs_layout_passes=False),
    )
    def _sc_stage(labels_hbm, c_hbm, present_hbm, cb_hbm, idx_v, rows_v,
                  pbuf, sem):
        wid = lax.axis_index("s") * 2 + lax.axis_index("c")
        base = wid * _BPW
        pltpu.sync_copy(labels_hbm.at[pl.ds(base, _BPW)], idx_v)
        copy = pltpu.async_copy(c_hbm.at[idx_v], rows_v, sem)

        zero16 = jnp.zeros((16,), jnp.float32)
        for i in range(_NCP // 16):
            pbuf[pl.ds(i * 16, 16)] = zero16
        ones16 = jnp.ones((16,), jnp.float32)
        for j in range(_BPW // 16):
            plsc.store_scatter(pbuf, [idx_v[pl.ds(j * 16, 16)]], ones16)
        pltpu.sync_copy(pbuf, present_hbm.at[wid])

        copy.wait()
        lane = jnp.full((16,), _LLANE, jnp.int32)
        for j in range(_BPW // 16):
            ridx = lax.broadcasted_iota(jnp.int32, (16,), 0) + j * 16
            vals = idx_v[pl.ds(j * 16, 16)].astype(jnp.float32)
            plsc.store_scatter(rows_v, [ridx, lane], vals)
        pltpu.sync_copy(rows_v, cb_hbm.at[pl.ds(base, _BPW)])

    return _sc_stage


def _sc_part(labels, cwide):
    return _sc_kernels()(labels, cwide)


def kernel(x, labels, centers):
    c = centers.reshape(_NC, _D)
    cpad = jnp.pad(c, ((0, _NCP - _NC), (0, 0)))
    cwide = jnp.pad(c, ((0, _NCP - _NC), (0, _DW - _D)))
    presentp, cbw = _sc_part(labels, cwide)
    sums = _tc_call(x, cpad, cpad, cbw, presentp, presentp)
    intraclass = sums[0] / (_B * _D * 2.0)
    triplet = sums[1] / (2.0 * _NC * _B)
    interclass = sums[2] / (_NC * _B * 2.0)
    return (intraclass, interclass, triplet)


# speedup vs baseline: 1.2435x; 1.0151x over previous
import functools

import jax
import jax.numpy as jnp
from jax import lax
from jax.experimental import pallas as pl
from jax.experimental.pallas import tpu as pltpu
from jax.experimental.pallas import tpu_sc as plsc

_B = 4096
_D = 32
_DW = 128
_LLANE = 32
_NSUB = 3
_NC = 3000
_NCP = 3072
_MARGIN = 1.0
_BX = 1024
_BC = 768
_BD = 384
_NS1 = _B // _BX
_KPAD = 40
_NEG = -1e30


def _augment(rows, b):
    n = rows.shape[0]
    parts = [rows + rows, b, jnp.ones((n, 1), jnp.float32)]
    if _KPAD > _D + 2:
        parts.append(jnp.zeros((n, _KPAD - _D - 2), jnp.float32))
    return jnp.concatenate(parts, axis=1)


def _tc_body(x_ref, cfull_ref, crows_ref, cb_ref, pfull_ref,
             pdiag_ref, out_ref, caug_s, q_s, acc_s):
    g = pl.program_id(0)
    ones_row = jnp.ones((1, _D), jnp.float32)

    @pl.when(g == 0)
    def _init():
        cfull = cfull_ref[:, :_D]
        c2_full = lax.dot_general(ones_row, cfull * cfull,
                                  (((1,), (1,)), ((), ())),
                                  preferred_element_type=jnp.float32)
        present = jnp.max(pfull_ref[...], axis=0, keepdims=True)
        colid = lax.broadcasted_iota(jnp.int32, (1, _NCP), 1)
        colcls = colid // _NSUB
        q_s[...] = jnp.where(present > 0.5, colcls, -1)
        negc2 = jnp.where(colid < _NC, -c2_full, _NEG)
        cparts = [cfull, jnp.ones((_NCP, 1), jnp.float32),
                  negc2.reshape(_NCP, 1)]
        if _KPAD > _D + 2:
            cparts.append(jnp.zeros((_NCP, _KPAD - _D - 2), jnp.float32))
        caug_s[...] = jnp.concatenate(cparts, axis=1)
        out_ref[0] = 0.0
        out_ref[1] = 0.0
        out_ref[2] = 0.0
        out_ref[3] = 0.0

    caug = caug_s[...]
    q = q_s[...]

    xb = x_ref[...]
    cbw = cb_ref[...]
    cbb = cbw[:, :_D]
    lab = cbw[:, _LLANE:_LLANE + 1].astype(jnp.int32)
    diff = xb - cbb
    intra = jnp.sum(diff * diff, axis=1, keepdims=True)
    x2 = jnp.sum(xb * xb, axis=1, keepdims=True)
    b = _MARGIN + intra - x2
    xaug = _augment(xb, b)
    t = lax.dot_general(xaug, caug, (((1,), (1,)), ((), ())),
                        preferred_element_type=jnp.float32)
    lcls = lab // _NSUB
    out_ref[0] += jnp.sum(intra)
    rpart = jnp.sum(jnp.where(q == lcls, 0.0, jnp.maximum(t, 0.0)),
                    axis=0, keepdims=True)
    @pl.when(g == 0)
    def _acc_init1():
        acc_s[0:1, :] = rpart
    @pl.when(g > 0)
    def _acc_add1():
        acc_s[0:1, :] += rpart

    cr_full = crows_ref[:, :_D]
    corr_sum = jnp.float32(0.0)
    for h in range(_BC // _BD):
        cr = cr_full[h * _BD:(h + 1) * _BD]
        c2col = jnp.sum(cr * cr, axis=1, keepdims=True)
        base = g * _BC + h * _BD
        rowid = lax.broadcasted_iota(jnp.int32, (_BD, 1), 0) + base
        rowcls = rowid // _NSUB
        colid_d = lax.broadcasted_iota(jnp.int32, (1, _BD), 1) + base
        colcls_d = colid_d // _NSUB
        pd = jnp.max(pdiag_ref[:, h * _BD:(h + 1) * _BD], axis=0,
                     keepdims=True)
        crsq = cr * cr
        c2row_d = lax.dot_general(ones_row, crsq, (((1,), (1,)), ((), ())),
                                  preferred_element_type=jnp.float32)
        ccd = lax.dot_general(cr, cr, (((1,), (1,)), ((), ())),
                              preferred_element_type=jnp.float32)
        dd = c2col + c2row_d - 2.0 * ccd

        samecls_d = rowcls == colcls_d
        eye = rowid == colid_d
        pdb = jnp.broadcast_to(pd, (_BD, _BD))
        pcol = jnp.max(jnp.where(eye, pdb, 0.0), axis=1,
                       keepdims=True)
        colm = jnp.max(jnp.where(samecls_d & (pcol > 0.5), dd, _NEG),
                       axis=0, keepdims=True)
        colmb = jnp.broadcast_to(colm, (_BD, _BD))
        dmax = jnp.max(jnp.where(samecls_d & (pdb > 0.5), colmb, _NEG),
                       axis=1, keepdims=True)
        cnt = jnp.sum(jnp.where(samecls_d, pdb, 0.0), axis=1,
                      keepdims=True)
        care = (cnt > 1.5).astype(jnp.float32)
        w = care * pcol

        b2 = jnp.where(w > 0.5, _MARGIN + dmax - c2col, _NEG)
        craug = _augment(cr, b2)
        t2 = lax.dot_general(craug, caug, (((1,), (1,)), ((), ())),
                             preferred_element_type=jnp.float32)
        t2d = 2.0 * ccd + b2 - c2row_d
        qd = jnp.where(pd > 0.5, colcls_d, -1)
        corr = jnp.where(qd == rowcls, jnp.maximum(t2d, 0.0), 0.0)
        corr_sum = corr_sum - jnp.sum(corr)
        r2part = jnp.sum(jnp.maximum(t2, 0.0), axis=0, keepdims=True)
        if h == 0:
            r2acc = r2part
        else:
            r2acc = r2acc + r2part

    out_ref[2] += corr_sum
    @pl.when(g == 0)
    def _acc_init2():
        acc_s[1:2, :] = r2acc
    @pl.when(g > 0)
    def _acc_add2():
        acc_s[1:2, :] += r2acc

    @pl.when(g == _NS1 - 1)
    def _final():
        out_ref[1] += jnp.sum(acc_s[0:1, :])
        out_ref[2] += jnp.sum(acc_s[1:2, :])


def _s1_map(g):
    return (g, 0)


def _s2_map(g):
    return (g, 0)


def _pdiag_map(g):
    return (0, g)


_TC_KW = dict(
    grid=(_NS1,),
    in_specs=[
        pl.BlockSpec((_BX, _D), _s1_map),
        pl.BlockSpec((_NCP, _DW), lambda g: (0, 0)),
        pl.BlockSpec((_BC, _DW), _s2_map),
        pl.BlockSpec((_BX, _DW), _s1_map),
        pl.BlockSpec((32, _NCP), lambda g: (0, 0)),
        pl.BlockSpec((32, _BC), _pdiag_map),
    ],
    out_specs=pl.BlockSpec(memory_space=pltpu.SMEM),
    out_shape=jax.ShapeDtypeStruct((4,), jnp.float32),
    scratch_shapes=[
        pltpu.VMEM((_NCP, _KPAD), jnp.float32),
        pltpu.VMEM((1, _NCP), jnp.int32),
        pltpu.VMEM((2, _NCP), jnp.float32),
    ],
    compiler_params=pltpu.CompilerParams(
        dimension_semantics=("arbitrary",)),
)

_tc_call = pl.pallas_call(_tc_body, **_TC_KW)


_NW = 32
_BPW = _B // _NW


@functools.cache
def _sc_kernels():
    mesh = plsc.VectorSubcoreMesh(core_axis_name="c", subcore_axis_name="s")

    @functools.partial(
        pl.kernel,
        mesh=mesh,
        out_type=[
            jax.ShapeDtypeStruct((_NW, _NCP), jnp.float32),
            jax.ShapeDtypeStruct((_B, _DW), jnp.float32),
        ],
        scratch_types=[
            pltpu.VMEM((_BPW,), jnp.int32),
            pltpu.VMEM((_BPW, _DW), jnp.float32),
            pltpu.VMEM((_NCP,), jnp.float32),
            pltpu.SemaphoreType.DMA,
        ],
        compiler_params=pltpu.CompilerParams(needs_layout_passes=False),
    )
    def _sc_stage(labels_hbm, c_hbm, present_hbm, cb_hbm, idx_v, rows_v,
                  pbuf, sem):
        wid = lax.axis_index("s") * 2 + lax.axis_index("c")
        base = wid * _BPW
        pltpu.sync_copy(labels_hbm.at[pl.ds(base, _BPW)], idx_v)
        copy = pltpu.async_copy(c_hbm.at[idx_v], rows_v, sem)

        zero16 = jnp.zeros((16,), jnp.float32)
        for i in range(_NCP // 16):
            pbuf[pl.ds(i * 16, 16)] = zero16
        ones16 = jnp.ones((16,), jnp.float32)
        for j in range(_BPW // 16):
            plsc.store_scatter(pbuf, [idx_v[pl.ds(j * 16, 16)]], ones16)
        pltpu.sync_copy(pbuf, present_hbm.at[wid])

        copy.wait()
        lane = jnp.full((16,), _LLANE, jnp.int32)
        for j in range(_BPW // 16):
            ridx = lax.broadcasted_iota(jnp.int32, (16,), 0) + j * 16
            vals = idx_v[pl.ds(j * 16, 16)].astype(jnp.float32)
            plsc.store_scatter(rows_v, [ridx, lane], vals)
        pltpu.sync_copy(rows_v, cb_hbm.at[pl.ds(base, _BPW)])

    return _sc_stage


def _sc_part(labels, cwide):
    return _sc_kernels()(labels, cwide)


def kernel(x, labels, centers):
    c = centers.reshape(_NC, _D)
    cwide = jnp.pad(c, ((0, _NCP - _NC), (0, _DW - _D)))
    presentp, cbw = _sc_part(labels, cwide)
    sums = _tc_call(x, cwide, cwide, cbw, presentp, presentp)
    intraclass = sums[0] / (_B * _D * 2.0)
    triplet = sums[1] / (2.0 * _NC * _B)
    interclass = sums[2] / (_NC * _B * 2.0)
    return (intraclass, interclass, triplet)
